# baseline (speedup 1.0000x reference)
import jax
import jax.numpy as jnp
from jax.experimental import pallas as pl
from jax.experimental.pallas import tpu as pltpu

_N = 1048576
_ROWS = _N // 128
_CROWS = 256
_NCHUNK = _ROWS // _CROWS


def _tc_body(p_hbm, m_hbm, o_ref, pbuf, mbuf, psem, msem):
    for c in range(_NCHUNK):
        pltpu.make_async_copy(
            p_hbm.at[pl.ds(c * _CROWS, _CROWS), :], pbuf.at[c], psem.at[c]
        ).start()
        pltpu.make_async_copy(
            m_hbm.at[pl.ds(c * _CROWS, _CROWS), :], mbuf.at[c], msem.at[c]
        ).start()

    acc = None
    for c in range(_NCHUNK):
        pltpu.make_async_copy(
            p_hbm.at[pl.ds(c * _CROWS, _CROWS), :], pbuf.at[c], psem.at[c]
        ).wait()
        pltpu.make_async_copy(
            m_hbm.at[pl.ds(c * _CROWS, _CROWS), :], mbuf.at[c], msem.at[c]
        ).wait()
        for k in range(0, _CROWS, 64):
            m = mbuf[c, pl.ds(k, 64), :]
            p = pbuf[c, pl.ds(k, 64), :]
            pm = jnp.where(m != 0.0, p, 0.0)
            contrib = jnp.exp(-0.5 * m * m) * pm
            acc = contrib if acc is None else acc + contrib

    while acc.shape[0] > 8:
        h = acc.shape[0] // 2
        acc = acc[:h] + acc[h:]
    o_ref[0, 0] = jnp.sum(acc) * (-1.0 / _N)


def kernel(preds, margin):
    p2 = preds.reshape(_ROWS, 128)
    m2 = margin.reshape(_ROWS, 128)
    out = pl.pallas_call(
        _tc_body,
        in_specs=[
            pl.BlockSpec(memory_space=pl.ANY),
            pl.BlockSpec(memory_space=pl.ANY),
        ],
        out_specs=pl.BlockSpec(memory_space=pltpu.SMEM),
        out_shape=jax.ShapeDtypeStruct((1, 1), jnp.float32),
        scratch_shapes=[
            pltpu.VMEM((_NCHUNK, _CROWS, 128), jnp.float32),
            pltpu.VMEM((_NCHUNK, _CROWS, 128), jnp.float32),
            pltpu.SemaphoreType.DMA((_NCHUNK,)),
            pltpu.SemaphoreType.DMA((_NCHUNK,)),
        ],
    )(p2, m2)
    return out[0, 0]


# speedup vs baseline: 1.1029x; 1.1029x over previous
import jax
import jax.numpy as jnp
from jax.experimental import pallas as pl
from jax.experimental.pallas import tpu as pltpu

_N = 1048576
_ROWS = _N // 128
_CROWS = 1024
_NCHUNK = _ROWS // _CROWS


def _tc_body(p_hbm, m_hbm, o_ref, pbuf, mbuf, psem, msem):
    for c in range(_NCHUNK):
        pltpu.make_async_copy(
            p_hbm.at[pl.ds(c * _CROWS, _CROWS), :], pbuf.at[c], psem.at[c]
        ).start()
        pltpu.make_async_copy(
            m_hbm.at[pl.ds(c * _CROWS, _CROWS), :], mbuf.at[c], msem.at[c]
        ).start()

    acc = None
    for c in range(_NCHUNK):
        pltpu.make_async_copy(
            p_hbm.at[pl.ds(c * _CROWS, _CROWS), :], pbuf.at[c], psem.at[c]
        ).wait()
        pltpu.make_async_copy(
            m_hbm.at[pl.ds(c * _CROWS, _CROWS), :], mbuf.at[c], msem.at[c]
        ).wait()
        for k in range(0, _CROWS, 64):
            m = mbuf[c, pl.ds(k, 64), :]
            p = pbuf[c, pl.ds(k, 64), :]
            pm = jnp.where(m != 0.0, p, 0.0)
            contrib = jnp.exp(-0.5 * m * m) * pm
            acc = contrib if acc is None else acc + contrib

    while acc.shape[0] > 8:
        h = acc.shape[0] // 2
        acc = acc[:h] + acc[h:]
    o_ref[0, 0] = jnp.sum(acc) * (-1.0 / _N)


def kernel(preds, margin):
    p2 = preds.reshape(_ROWS, 128)
    m2 = margin.reshape(_ROWS, 128)
    out = pl.pallas_call(
        _tc_body,
        in_specs=[
            pl.BlockSpec(memory_space=pl.ANY),
            pl.BlockSpec(memory_space=pl.ANY),
        ],
        out_specs=pl.BlockSpec(memory_space=pltpu.SMEM),
        out_shape=jax.ShapeDtypeStruct((1, 1), jnp.float32),
        scratch_shapes=[
            pltpu.VMEM((_NCHUNK, _CROWS, 128), jnp.float32),
            pltpu.VMEM((_NCHUNK, _CROWS, 128), jnp.float32),
            pltpu.SemaphoreType.DMA((_NCHUNK,)),
            pltpu.SemaphoreType.DMA((_NCHUNK,)),
        ],
    )(p2, m2)
    return out[0, 0]
